# baseline (speedup 1.0000x reference)
import jax
import jax.numpy as jnp
from jax.experimental import pallas as pl
from jax.experimental.pallas import tpu as pltpu

EPS = 1e-5
B = 1024
NPF, NSV = 50, 5
NV = 55
NP = 56
K = 7
M = K * NP
BB = 32
F32 = jnp.float32


def _row_mask(shape_rows):
    n = jax.lax.broadcasted_iota(jnp.int32, (1, shape_rows, 1), 1) % NP
    return (n < NV).astype(F32)


def _acc(ref, val, i):
    @pl.when(i == 0)
    def _():
        ref[...] = jnp.zeros_like(ref)
    ref[...] += val


def _masked_stats(x, mask, ref, i):
    s = jnp.sum(x * mask, axis=(0, 1))[None, :]
    q = jnp.sum(x * x * mask, axis=(0, 1))[None, :]
    _acc(ref, jnp.concatenate([s, q], axis=0), i)


def _mv(stat, n):
    m = stat[0:1, :] / n
    v = stat[1:2, :] / n - m * m
    return m, jax.lax.rsqrt(v + EPS)


def _k1(pf_ref, sv_ref, pfs_ref, pfS_ref, svs_ref, svS_ref):
    i = pl.program_id(0)
    for x_ref, s_ref, S_ref, cdim, ndim in ((pf_ref, pfs_ref, pfS_ref, 22, NPF),
                                            (sv_ref, svs_ref, svS_ref, 12, NSV)):
        xt = jnp.swapaxes(x_ref[...], 1, 2)
        x2 = xt.reshape(BB * ndim, cdim)
        _acc(s_ref, jnp.sum(x2, axis=0)[None, :], i)
        S = jax.lax.dot_general(x2, x2, (((0,), (0,)), ((), ())),
                                preferred_element_type=F32)
        _acc(S_ref, S, i)


def _feature_conv(x_ref, s_ref, S_ref, w_ref, cdim, ndim):
    n_tot = B * ndim
    ssum = s_ref[...]
    S = S_ref[...] / n_tot
    m = ssum / n_tot
    eye = (jax.lax.broadcasted_iota(jnp.int32, S.shape, 0)
           == jax.lax.broadcasted_iota(jnp.int32, S.shape, 1)).astype(F32)
    var1 = jnp.sum(S * eye, axis=0)[None, :] - m * m
    inv1 = jax.lax.rsqrt(var1 + EPS)
    C = (S - m.T * m) * inv1.T * inv1
    w = w_ref[...]
    var2 = jnp.sum((w @ C) * w, axis=1)[None, :]
    inv2 = jax.lax.rsqrt(var2 + EPS)
    xt = jnp.swapaxes(x_ref[...], 1, 2)
    f1 = (xt - m[None, :, :]) * inv1[None, :, :]
    y = jax.lax.dot_general(f1.reshape(BB * ndim, cdim), w,
                            (((1,), (1,)), ((), ())), preferred_element_type=F32)
    return jax.nn.relu(y * inv2).reshape(BB, ndim, 32)


def _knn_idx(d2):
    iota_n = jax.lax.broadcasted_iota(jnp.int32, (BB, NP, NP), 1)
    iota_m = jax.lax.broadcasted_iota(jnp.int32, (BB, NP, NP), 2)
    bits = jax.lax.bitcast_convert_type(jnp.maximum(d2, 0.0), jnp.int32)
    packed = (bits & jnp.int32(~63)) | iota_m
    big = jnp.int32(0x7FFFFFFF)
    packed = jnp.where((iota_n == iota_m) | (iota_m >= NV), big, packed)
    sels = []
    for _ in range(K):
        cmin = jnp.min(packed, axis=2)
        sels.append(cmin & 63)
        packed = jnp.where(packed == cmin[:, :, None], big, packed)
    return jnp.concatenate(sels, axis=1)


def _gather_cf(table_cl, idxlane, cdim):
    tcf = jnp.swapaxes(table_cl, 1, 2)
    idxb = jnp.broadcast_to(idxlane[:, None, :], (BB, cdim, M))
    return jnp.swapaxes(jnp.take_along_axis(tcf, idxb, axis=2), 1, 2)


def _k2(pf_ref, sv_ref, pfp_ref, svp_ref, wpf_ref, wsv_ref,
        pfs_ref, pfS_ref, svs_ref, svS_ref,
        feat_ref, nb_ref, fs_ref):
    i = pl.program_id(0)
    p1 = _feature_conv(pf_ref, pfs_ref, pfS_ref, wpf_ref, 22, NPF)
    p2 = _feature_conv(sv_ref, svs_ref, svS_ref, wsv_ref, 12, NSV)
    feat = jnp.concatenate([p1, p2, jnp.zeros((BB, 1, 32), F32)], axis=1)
    feat_ref[...] = feat
    _masked_stats(feat, 1.0, fs_ref, i)
    pts = jnp.concatenate([pfp_ref[...], svp_ref[...],
                           jnp.zeros((BB, 2, 1), F32)], axis=2)
    px, py = pts[:, 0, :], pts[:, 1, :]
    dx = px[:, :, None] - px[:, None, :]
    dy = py[:, :, None] - py[:, None, :]
    idxlane = _knn_idx(dx * dx + dy * dy)
    nb_ref[...] = _gather_cf(feat, idxlane, 32)


def _rep7(u):
    return jnp.concatenate([u] * K, axis=1)


def _h1_raw(feat_n, nbn, w0, co):
    ci = feat_n.shape[2]
    wa, wb = w0[:, :ci], w0[:, ci:]
    u = jax.lax.dot_general(feat_n.reshape(BB * NP, ci), wa - wb,
                            (((1,), (1,)), ((), ())), preferred_element_type=F32)
    v = jax.lax.dot_general(nbn.reshape(BB * M, ci), wb,
                            (((1,), (1,)), ((), ())), preferred_element_type=F32)
    return _rep7(u.reshape(BB, NP, co)) + v.reshape(BB, M, co)


def _mm(x, w):
    ci, co = x.shape[2], w.shape[0]
    y = jax.lax.dot_general(x.reshape(BB * M, ci), w,
                            (((1,), (1,)), ((), ())), preferred_element_type=F32)
    return y.reshape(BB, M, co)


def _bnrelu(x, stat, n):
    m, inv = _mv(stat, n)
    return jax.nn.relu((x - m[None, :, :]) * inv[None, :, :])


NK = B * NV * K
N1 = B * NV


def _ec_stage(depth, feat_n, nbn, ws, stats):
    h = _h1_raw(feat_n, nbn, ws[0], ws[0].shape[0])
    for d in range(depth):
        h = _bnrelu(h, stats[d], NK)
        h = _mm(h, ws[d + 1])
    return h


def _make_ec_stat_kernel(depth, ci, ws_count, normalize_table):
    def kern(feat_ref, nb_ref, fs_ref, *rest):
        i = pl.program_id(0)
        ws = [rest[j][...] for j in range(ws_count)]
        stats = [rest[ws_count + j][...] for j in range(depth)]
        out_ref = rest[ws_count + depth]
        mask56 = _row_mask(NP)
        mask = _row_mask(M)
        feat = feat_ref[...]
        nb = nb_ref[...]
        if normalize_table:
            m, inv = _mv(fs_ref[...], N1)
            feat = ((feat - m[None, :, :]) * inv[None, :, :]) * mask56
            nb = (nb - m[None, :, :]) * inv[None, :, :]
        h = _ec_stage(depth, feat, nb, ws, stats)
        _masked_stats(h, mask, out_ref, i)
    return kern


def _k6(feat_ref, nb_ref, fs_ref, w0_ref, w1_ref, w2_ref, scw_ref,
        s1_ref, s2_ref, s3_ref,
        out1_ref, nb2_ref, scs_ref):
    i = pl.program_id(0)
    mask56 = _row_mask(NP)
    m, inv = _mv(fs_ref[...], N1)
    feat = feat_ref[...]
    fts0 = ((feat - m[None, :, :]) * inv[None, :, :]) * mask56
    nbn = (nb_ref[...] - m[None, :, :]) * inv[None, :, :]
    ws = [w0_ref[...], w1_ref[...], w2_ref[...]]
    stats = [s1_ref[...], s2_ref[...], s3_ref[...]]
    h = _ec_stage(2, fts0, nbn, ws, stats)
    h = _bnrelu(h, stats[2], NK)
    mk = sum(h[:, k * NP:(k + 1) * NP, :] for k in range(K)) * (1.0 / K)
    out1 = jax.nn.relu(fts0 + mk) * mask56
    out1_ref[...] = out1
    sc_raw = jax.lax.dot_general(out1.reshape(BB * NP, 32), scw_ref[...],
                                 (((1,), (1,)), ((), ())),
                                 preferred_element_type=F32).reshape(BB, NP, 64)
    _masked_stats(sc_raw, mask56, scs_ref, i)
    xx = jnp.sum(out1 * out1, axis=2)
    g = jax.lax.dot_general(out1, jnp.swapaxes(out1, 1, 2),
                            (((2,), (1,)), ((0,), (0,))),
                            preferred_element_type=F32)
    d2 = xx[:, :, None] + xx[:, None, :] - 2.0 * g
    idxlane = _knn_idx(d2)
    nb2_ref[...] = _gather_cf(out1, idxlane, 32)


def _k10(out1_ref, nb2_ref, w0_ref, w1_ref, w2_ref, scw_ref, fw_ref,
         scs_ref, s1_ref, s2_ref, s3_ref,
         out2_ref, fus_ref):
    i = pl.program_id(0)
    mask56 = _row_mask(NP)
    out1 = out1_ref[...]
    nb = nb2_ref[...]
    ws = [w0_ref[...], w1_ref[...], w2_ref[...]]
    stats = [s1_ref[...], s2_ref[...], s3_ref[...]]
    h = _ec_stage(2, out1, nb, ws, stats)
    h = _bnrelu(h, stats[2], NK)
    mk = sum(h[:, k * NP:(k + 1) * NP, :] for k in range(K)) * (1.0 / K)
    sc_raw = jax.lax.dot_general(out1.reshape(BB * NP, 32), scw_ref[...],
                                 (((1,), (1,)), ((), ())),
                                 preferred_element_type=F32).reshape(BB, NP, 64)
    m, inv = _mv(scs_ref[...], N1)
    sc = (sc_raw - m[None, :, :]) * inv[None, :, :]
    out2 = jax.nn.relu(sc + mk) * mask56
    out2_ref[...] = out2
    fw = fw_ref[...]
    fa, fb = fw[:, :32], fw[:, 32:]
    fr = (jax.lax.dot_general(out1.reshape(BB * NP, 32), fa,
                              (((1,), (1,)), ((), ())), preferred_element_type=F32)
          + jax.lax.dot_general(out2.reshape(BB * NP, 64), fb,
                                (((1,), (1,)), ((), ()), ),
                                preferred_element_type=F32)).reshape(BB, NP, 128)
    _masked_stats(fr, mask56, fus_ref, i)


def _k11(out1_ref, out2_ref, fw_ref, fus_ref,
         fc1w_ref, fc1b_ref, fc2w_ref, fc2b_ref, o_ref):
    mask56 = _row_mask(NP)
    out1, out2 = out1_ref[...], out2_ref[...]
    fw = fw_ref[...]
    fa, fb = fw[:, :32], fw[:, 32:]
    fr = (jax.lax.dot_general(out1.reshape(BB * NP, 32), fa,
                              (((1,), (1,)), ((), ())), preferred_element_type=F32)
          + jax.lax.dot_general(out2.reshape(BB * NP, 64), fb,
                                (((1,), (1,)), ((), ())),
                                preferred_element_type=F32)).reshape(BB, NP, 128)
    m, inv = _mv(fus_ref[...], N1)
    fused = jax.nn.relu((fr - m[None, :, :]) * inv[None, :, :]) * mask56
    pooled = jnp.sum(fused, axis=1) * (1.0 / NV)
    x1 = jax.nn.relu(jax.lax.dot_general(pooled, fc1w_ref[...],
                                         (((1,), (1,)), ((), ())),
                                         preferred_element_type=F32)
                     + fc1b_ref[...])
    o_ref[...] = jax.lax.dot_general(x1, fc2w_ref[...],
                                     (((1,), (1,)), ((), ())),
                                     preferred_element_type=F32) + fc2b_ref[...]


def _spec(shape, blocked_dim0=True):
    if blocked_dim0:
        zeros = (0,) * (len(shape) - 1)
        return pl.BlockSpec(shape, lambda i: (i,) + zeros)
    return pl.BlockSpec(shape, lambda i: (0,) * len(shape))


def _full(shape):
    return _spec(shape, blocked_dim0=False)


def kernel(pf_points, pf_features, pf_mask, sv_points, sv_features, sv_mask,
           pf_conv_w, sv_conv_w, ec1_w0, ec1_w1, ec1_w2,
           ec2_w0, ec2_w1, ec2_w2, ec2_sc_w, fusion_w,
           fc1_w, fc1_b, fc2_w, fc2_b):
    nb_blocks = B // BB
    grid = (nb_blocks,)

    def call(kern, in_arrays, in_specs, out_shapes, out_specs):
        return pl.pallas_call(
            kern, grid=grid, in_specs=in_specs,
            out_shape=[jax.ShapeDtypeStruct(s, d) for s, d in out_shapes],
            out_specs=out_specs)(*in_arrays)

    pf_s, pf_S, sv_s, sv_S = call(
        _k1,
        [pf_features, sv_features],
        [_spec((BB, 22, NPF)), _spec((BB, 12, NSV))],
        [((1, 22), F32), ((22, 22), F32), ((1, 12), F32), ((12, 12), F32)],
        [_full((1, 22)), _full((22, 22)), _full((1, 12)), _full((12, 12))])

    feat, nb1_cf, f_stat = call(
        _k2,
        [pf_features, sv_features, pf_points, sv_points, pf_conv_w, sv_conv_w,
         pf_s, pf_S, sv_s, sv_S],
        [_spec((BB, 22, NPF)), _spec((BB, 12, NSV)), _spec((BB, 2, NPF)),
         _spec((BB, 2, NSV)), _full((32, 22)), _full((32, 12)),
         _full((1, 22)), _full((22, 22)), _full((1, 12)), _full((12, 12))],
        [((B, NP, 32), F32), ((B, M, 32), F32), ((2, 32), F32)],
        [_spec((BB, NP, 32)), _spec((BB, M, 32)), _full((2, 32))])

    ec1_ws = [ec1_w0, ec1_w1, ec1_w2]
    ec1_w_specs = [_full((32, 64)), _full((32, 32)), _full((32, 32))]
    stats1 = []
    for depth in range(3):
        kern = _make_ec_stat_kernel(depth, 32, depth + 1, True)
        (st,) = call(
            kern,
            [feat, nb1_cf, f_stat] + ec1_ws[:depth + 1] + stats1,
            [_spec((BB, NP, 32)), _spec((BB, M, 32)), _full((2, 32))]
            + ec1_w_specs[:depth + 1] + [_full((2, 32))] * depth,
            [((2, 32), F32)], [_full((2, 32))])
        stats1.append(st)

    out1, nb2_cf, sc_stat = call(
        _k6,
        [feat, nb1_cf, f_stat, ec1_w0, ec1_w1, ec1_w2, ec2_sc_w] + stats1,
        [_spec((BB, NP, 32)), _spec((BB, M, 32)), _full((2, 32)),
         _full((32, 64)), _full((32, 32)), _full((32, 32)), _full((64, 32))]
        + [_full((2, 32))] * 3,
        [((B, NP, 32), F32), ((B, M, 32), F32), ((2, 64), F32)],
        [_spec((BB, NP, 32)), _spec((BB, M, 32)), _full((2, 64))])

    ec2_ws = [ec2_w0, ec2_w1, ec2_w2]
    ec2_w_specs = [_full((64, 64))] * 3
    stats2 = []
    for depth in range(3):
        kern = _make_ec_stat_kernel(depth, 32, depth + 1, False)
        (st,) = call(
            kern,
            [out1, nb2_cf, f_stat] + ec2_ws[:depth + 1] + stats2,
            [_spec((BB, NP, 32)), _spec((BB, M, 32)), _full((2, 32))]
            + ec2_w_specs[:depth + 1] + [_full((2, 64))] * depth,
            [((2, 64), F32)], [_full((2, 64))])
        stats2.append(st)

    out2, fus_stat = call(
        _k10,
        [out1, nb2_cf, ec2_w0, ec2_w1, ec2_w2, ec2_sc_w, fusion_w,
         sc_stat] + stats2,
        [_spec((BB, NP, 32)), _spec((BB, M, 32)), _full((64, 64)),
         _full((64, 64)), _full((64, 64)), _full((64, 32)), _full((128, 96)),
         _full((2, 64))] + [_full((2, 64))] * 3,
        [((B, NP, 64), F32), ((2, 128), F32)],
        [_spec((BB, NP, 64)), _full((2, 128))])

    (out,) = call(
        _k11,
        [out1, out2, fusion_w, fus_stat,
         fc1_w, fc1_b.reshape(1, 128), fc2_w, fc2_b.reshape(1, 4)],
        [_spec((BB, NP, 32)), _spec((BB, NP, 64)), _full((128, 96)),
         _full((2, 128)), _full((128, 128)), _full((1, 128)),
         _full((4, 128)), _full((1, 4))],
        [((B, 4), F32)],
        [_spec((BB, 4))])
    return out


# speedup vs baseline: 1.1362x; 1.1362x over previous
import jax
import jax.numpy as jnp
from jax.experimental import pallas as pl
from jax.experimental.pallas import tpu as pltpu

EPS = 1e-5
B = 1024
NPF, NSV = 50, 5
NV = 55
NP = 56
K = 7
M = K * NP
F32 = jnp.float32
NK = B * NV * K
N1 = B * NV


def _row_mask(rows):
    n = jax.lax.broadcasted_iota(jnp.int32, (1, rows, 1), 1) % NP
    return (n < NV).astype(F32)


def _acc(ref, val, i):
    @pl.when(i == 0)
    def _():
        ref[...] = jnp.zeros_like(ref)
    ref[...] += val


def _dot(a, b):
    return jax.lax.dot_general(a, b, (((1,), (1,)), ((), ())),
                               preferred_element_type=F32)


def _mom(a, b):
    return jax.lax.dot_general(a, b, (((0,), (0,)), ((), ())),
                               preferred_element_type=F32)


def _s8(x2d):
    r, c = x2d.shape
    return jnp.sum(x2d.reshape(r // 8, 8, c), axis=0)


def _colsum(x):
    return jnp.sum(x, axis=0, keepdims=True)


def _diag(S):
    eye = (jax.lax.broadcasted_iota(jnp.int32, S.shape, 0)
           == jax.lax.broadcasted_iota(jnp.int32, S.shape, 1)).astype(F32)
    return jnp.sum(S * eye, axis=0)[None, :]


def _zstats(wab, wb, sff, sfn, snn, muf, mun):
    mean = _dot(muf, wab) + _dot(mun, wb)
    d1 = jnp.sum((wab @ sff) * wab, axis=1)[None, :]
    d2 = jnp.sum((wab @ sfn) * wb, axis=1)[None, :]
    d4 = jnp.sum((wb @ snn) * wb, axis=1)[None, :]
    var = d1 + 2.0 * d2 + d4 - mean * mean
    return mean, jax.lax.rsqrt(var + EPS)


def _lin_stats(w, S, mu):
    mean = _dot(mu, w)
    e2 = jnp.sum((w @ S) * w, axis=1)[None, :]
    return mean, jax.lax.rsqrt(e2 - mean * mean + EPS)


def _unpack_gm(gm, c):
    A = gm[0:c, :] / NK
    Bm = gm[c:2 * c, :] / NK
    D = gm[2 * c:3 * c, :] / NK
    muf = _colsum(gm[3 * c:3 * c + 8, :]) / NK
    mun = _colsum(gm[3 * c + 8:3 * c + 16, :]) / NK
    return A, Bm, D, muf, mun


def _unpack_gh(gh, c):
    S = gh[0:c, :] / NK
    mu = _colsum(gh[c:c + 8, :]) / NK
    return S, mu


def _knn_idx(d2):
    iota_n = jax.lax.broadcasted_iota(jnp.int32, d2.shape, 1)
    iota_m = jax.lax.broadcasted_iota(jnp.int32, d2.shape, 2)
    bits = jax.lax.bitcast_convert_type(jnp.maximum(d2, 0.0), jnp.int32)
    packed = (bits & jnp.int32(~63)) | iota_m
    big = jnp.int32(0x7FFFFFFF)
    packed = jnp.where((iota_n == iota_m) | (iota_m >= NV), big, packed)
    sels = []
    for _ in range(K):
        cmin = jnp.min(packed, axis=2)
        sels.append(cmin & 63)
        packed = jnp.where(packed == cmin[:, :, None], big, packed)
    return jnp.concatenate(sels, axis=1)


def _gather_cl(table_cl, idxlane, cdim):
    bb = table_cl.shape[0]
    tcf = jnp.swapaxes(table_cl, 1, 2)
    idxb = jnp.broadcast_to(idxlane[:, None, :], (bb, cdim, M))
    return jnp.swapaxes(jnp.take_along_axis(tcf, idxb, axis=2), 1, 2)


def _rep7(u):
    return jnp.concatenate([u] * K, axis=1)


def _ksum(x3, c):
    return sum(x3[:, k * NP:(k + 1) * NP, :] for k in range(K))


def _gather_moments(table, nbm, gm_ref, i):
    bb, _, c = table.shape
    t2 = table.reshape(bb * NP, c)
    nb2 = nbm.reshape(bb * M, c)
    nbsum2 = _ksum(nbm, c).reshape(bb * NP, c)
    A = 7.0 * _mom(t2, t2)
    Bm = _mom(t2, nbsum2)
    D = _mom(nb2, nb2)
    val = jnp.concatenate([A, Bm, D, 7.0 * _s8(t2), _s8(nb2)], axis=0)
    _acc(gm_ref, val, i)


def _h_moments(hm2, gh_ref, i):
    val = jnp.concatenate([_mom(hm2, hm2), _s8(hm2)], axis=0)
    _acc(gh_ref, val, i)


def _k1(pf_ref, sv_ref, pfs_ref, pfS_ref, svs_ref, svS_ref):
    i = pl.program_id(0)
    for x_ref, s_ref, S_ref, cdim, ndim in ((pf_ref, pfs_ref, pfS_ref, 22, NPF),
                                            (sv_ref, svs_ref, svS_ref, 12, NSV)):
        bb = x_ref.shape[0]
        xt = jnp.swapaxes(x_ref[...], 1, 2)
        x2 = xt.reshape(bb * ndim, cdim)
        _acc(s_ref, jnp.sum(x2, axis=0)[None, :], i)
        _acc(S_ref, _mom(x2, x2), i)


def _feature_conv(x_ref, s_ref, S_ref, w_ref, cdim, ndim):
    n_tot = B * ndim
    S = S_ref[...] / n_tot
    m = s_ref[...] / n_tot
    var1 = _diag(S) - m * m
    inv1 = jax.lax.rsqrt(var1 + EPS)
    C = (S - m.T * m) * inv1.T * inv1
    w = w_ref[...]
    var2 = jnp.sum((w @ C) * w, axis=1)[None, :]
    inv2 = jax.lax.rsqrt(var2 + EPS)
    bb = x_ref.shape[0]
    xt = jnp.swapaxes(x_ref[...], 1, 2)
    f1 = (xt - m[None, :, :]) * inv1[None, :, :]
    y = _dot(f1.reshape(bb * ndim, cdim), w)
    return jax.nn.relu(y * inv2).reshape(bb, ndim, 32)


def _k2(pf_ref, sv_ref, pfp_ref, svp_ref, wpf_ref, wsv_ref,
        pfs_ref, pfS_ref, svs_ref, svS_ref,
        feat_ref, nb_ref, gm_ref):
    i = pl.program_id(0)
    p1 = _feature_conv(pf_ref, pfs_ref, pfS_ref, wpf_ref, 22, NPF)
    p2 = _feature_conv(sv_ref, svs_ref, svS_ref, wsv_ref, 12, NSV)
    bb = p1.shape[0]
    feat = jnp.concatenate([p1, p2, jnp.zeros((bb, 1, 32), F32)], axis=1)
    feat_ref[...] = feat
    pts = jnp.concatenate([pfp_ref[...], svp_ref[...],
                           jnp.zeros((bb, 2, 1), F32)], axis=2)
    px, py = pts[:, 0, :], pts[:, 1, :]
    dx = px[:, :, None] - px[:, None, :]
    dy = py[:, :, None] - py[:, None, :]
    idxlane = _knn_idx(dx * dx + dy * dy)
    nbm = _gather_cl(feat, idxlane, 32) * _row_mask(M)
    nb_ref[...] = nbm
    _gather_moments(feat, nbm, gm_ref, i)


def _ec_weights(w0, inv_tbl):
    ci = w0.shape[1] // 2
    wa, wb = w0[:, :ci], w0[:, ci:]
    wab = wa - wb
    if inv_tbl is not None:
        wab = wab * inv_tbl
        wb = wb * inv_tbl
    return wab, wb


def _ec_chain(depth, feat, nb, ws, gm, gh_list, inv_tbl):
    bb, _, ci = feat.shape
    A, Bm, D, muf, mun = _unpack_gm(gm, ci)
    wab, wb = _ec_weights(ws[0], inv_tbl)
    mean, inv = _zstats(wab, wb, A, Bm, D, muf, mun)
    z = (_rep7(_dot(feat.reshape(bb * NP, ci), wab).reshape(bb, NP, -1))
         + _dot(nb.reshape(bb * M, ci), wb).reshape(bb, M, -1))
    h = jax.nn.relu((z - mean[None, :, :]) * inv[None, :, :])
    for d in range(depth):
        co = ws[d + 1].shape[0]
        S, mu = _unpack_gh(gh_list[d], h.shape[2])
        mean, inv = _lin_stats(ws[d + 1], S, mu)
        z = _dot(h.reshape(bb * M, h.shape[2]), ws[d + 1]).reshape(bb, M, co)
        h = jax.nn.relu((z - mean[None, :, :]) * inv[None, :, :])
    return h


def _tbl_inv(gm, c):
    A, _, _, muf, _ = _unpack_gm(gm, c)
    return muf, jax.lax.rsqrt(_diag(A) - muf * muf + EPS)


def _make_mid_kernel(depth, normalize_table):
    def kern(feat_ref, nb_ref, gm_ref, *rest):
        i = pl.program_id(0)
        nw = depth + 1
        ws = [rest[j][...] for j in range(nw)]
        ghs = [rest[nw + j][...] for j in range(depth)]
        out_ref = rest[nw + depth]
        gm = gm_ref[...]
        feat, nb = feat_ref[...], nb_ref[...]
        inv_tbl = _tbl_inv(gm, feat.shape[2])[1] if normalize_table else None
        h = _ec_chain(depth, feat, nb, ws, gm, ghs, inv_tbl)
        bb, _, co = h.shape
        hm2 = (h * _row_mask(M)).reshape(bb * M, co)
        _h_moments(hm2, out_ref, i)
    return kern


def _k6(feat_ref, nb_ref, gm_ref, w0_ref, w1_ref, w2_ref,
        g1_ref, g2_ref,
        out1_ref, nb2_ref, gm2_ref):
    i = pl.program_id(0)
    gm = gm_ref[...]
    feat, nb = feat_ref[...], nb_ref[...]
    bb = feat.shape[0]
    muf, inv_tbl = _tbl_inv(gm, 32)
    ws = [w0_ref[...], w1_ref[...], w2_ref[...]]
    h3 = _ec_chain(2, feat, nb, ws, gm, [g1_ref[...], g2_ref[...]], inv_tbl)
    mk = _ksum(h3, 32) * (1.0 / K)
    mask56 = _row_mask(NP)
    fts0 = (feat - muf[None, :, :]) * inv_tbl[None, :, :]
    out1 = jax.nn.relu(fts0 + mk) * mask56
    out1_ref[...] = out1
    xx = jnp.sum(out1 * out1, axis=2)
    g = jax.lax.dot_general(out1, jnp.swapaxes(out1, 1, 2),
                            (((2,), (1,)), ((0,), (0,))),
                            preferred_element_type=F32)
    d2 = xx[:, :, None] + xx[:, None, :] - 2.0 * g
    idxlane = _knn_idx(d2)
    nbm2 = _gather_cl(out1, idxlane, 32) * _row_mask(M)
    nb2_ref[...] = nbm2
    _gather_moments(out1, nbm2, gm2_ref, i)


def _k10(out1_ref, nb2_ref, gm2_ref, w0_ref, w1_ref, w2_ref, scw_ref,
         g1_ref, g2_ref,
         out2_ref, gf_ref):
    i = pl.program_id(0)
    gm2 = gm2_ref[...]
    out1, nb2 = out1_ref[...], nb2_ref[...]
    bb = out1.shape[0]
    ws = [w0_ref[...], w1_ref[...], w2_ref[...]]
    h3 = _ec_chain(2, out1, nb2, ws, gm2, [g1_ref[...], g2_ref[...]], None)
    mk = _ksum(h3, 64) * (1.0 / K)
    A2, _, _, mu1, _ = _unpack_gm(gm2, 32)
    scw = scw_ref[...]
    mean_sc, inv_sc = _lin_stats(scw, A2, mu1)
    sc = (_dot(out1.reshape(bb * NP, 32), scw).reshape(bb, NP, 64)
          - mean_sc[None, :, :]) * inv_sc[None, :, :]
    mask56 = _row_mask(NP)
    out2 = jax.nn.relu(sc + mk) * mask56
    out2_ref[...] = out2
    o12, o22 = out1.reshape(bb * NP, 32), out2.reshape(bb * NP, 64)
    val = jnp.concatenate([_mom(o12, o22), _mom(o22, o22), _s8(o22)], axis=0)
    _acc(gf_ref, val, i)


def _k11(out1_ref, out2_ref, gm2_ref, gf_ref, fw_ref,
         fc1w_ref, fc1b_ref, fc2w_ref, fc2b_ref, o_ref):
    out1, out2 = out1_ref[...], out2_ref[...]
    bb = out1.shape[0]
    gm2, gf = gm2_ref[...], gf_ref[...]
    A2, _, _, mu1, _ = _unpack_gm(gm2, 32)
    S11 = A2
    S12 = gf[0:32, :] / N1
    S22 = gf[32:96, :] / N1
    mu2 = _colsum(gf[96:104, :]) / N1
    fw = fw_ref[...]
    fa, fb = fw[:, :32], fw[:, 32:]
    mean = _dot(mu1, fa) + _dot(mu2, fb)
    d1 = jnp.sum((fa @ S11) * fa, axis=1)[None, :]
    d2 = jnp.sum((fa @ S12) * fb, axis=1)[None, :]
    d4 = jnp.sum((fb @ S22) * fb, axis=1)[None, :]
    inv = jax.lax.rsqrt(d1 + 2.0 * d2 + d4 - mean * mean + EPS)
    fr = (_dot(out1.reshape(bb * NP, 32), fa)
          + _dot(out2.reshape(bb * NP, 64), fb)).reshape(bb, NP, 128)
    fused = jax.nn.relu((fr - mean[None, :, :]) * inv[None, :, :]) * _row_mask(NP)
    pooled = jnp.sum(fused, axis=1) * (1.0 / NV)
    x1 = jax.nn.relu(_dot(pooled, fc1w_ref[...]) + fc1b_ref[...])
    o_ref[...] = _dot(x1, fc2w_ref[...]) + fc2b_ref[...]


def _spec(shape, blocked_dim0=True):
    if blocked_dim0:
        zeros = (0,) * (len(shape) - 1)
        return pl.BlockSpec(shape, lambda i: (i,) + zeros)
    return pl.BlockSpec(shape, lambda i: (0,) * len(shape))


def _full(shape):
    return _spec(shape, blocked_dim0=False)


def kernel(pf_points, pf_features, pf_mask, sv_points, sv_features, sv_mask,
           pf_conv_w, sv_conv_w, ec1_w0, ec1_w1, ec1_w2,
           ec2_w0, ec2_w1, ec2_w2, ec2_sc_w, fusion_w,
           fc1_w, fc1_b, fc2_w, fc2_b):

    def call(kern, bb, in_arrays, in_specs, out_shapes, out_specs):
        return pl.pallas_call(
            kern, grid=(B // bb,), in_specs=in_specs,
            out_shape=[jax.ShapeDtypeStruct(s, d) for s, d in out_shapes],
            out_specs=out_specs)(*in_arrays)

    b1 = 128
    pf_s, pf_S, sv_s, sv_S = call(
        _k1, b1,
        [pf_features, sv_features],
        [_spec((b1, 22, NPF)), _spec((b1, 12, NSV))],
        [((1, 22), F32), ((22, 22), F32), ((1, 12), F32), ((12, 12), F32)],
        [_full((1, 22)), _full((22, 22)), _full((1, 12)), _full((12, 12))])

    b2 = 64
    feat, nb1, gm1 = call(
        _k2, b2,
        [pf_features, sv_features, pf_points, sv_points, pf_conv_w, sv_conv_w,
         pf_s, pf_S, sv_s, sv_S],
        [_spec((b2, 22, NPF)), _spec((b2, 12, NSV)), _spec((b2, 2, NPF)),
         _spec((b2, 2, NSV)), _full((32, 22)), _full((32, 12)),
         _full((1, 22)), _full((22, 22)), _full((1, 12)), _full((12, 12))],
        [((B, NP, 32), F32), ((B, M, 32), F32), ((112, 32), F32)],
        [_spec((b2, NP, 32)), _spec((b2, M, 32)), _full((112, 32))])

    b3 = 64
    ec1_ws = [ec1_w0, ec1_w1, ec1_w2]
    ec1_w_specs = [_full((32, 64)), _full((32, 32)), _full((32, 32))]
    ghs1 = []
    for depth in range(2):
        kern = _make_mid_kernel(depth, True)
        (gh,) = call(
            kern, b3,
            [feat, nb1, gm1] + ec1_ws[:depth + 1] + ghs1,
            [_spec((b3, NP, 32)), _spec((b3, M, 32)), _full((112, 32))]
            + ec1_w_specs[:depth + 1] + [_full((40, 32))] * depth,
            [((40, 32), F32)], [_full((40, 32))])
        ghs1.append(gh)

    b6 = 32
    out1, nb2, gm2 = call(
        _k6, b6,
        [feat, nb1, gm1, ec1_w0, ec1_w1, ec1_w2] + ghs1,
        [_spec((b6, NP, 32)), _spec((b6, M, 32)), _full((112, 32)),
         _full((32, 64)), _full((32, 32)), _full((32, 32))]
        + [_full((40, 32))] * 2,
        [((B, NP, 32), F32), ((B, M, 32), F32), ((112, 32), F32)],
        [_spec((b6, NP, 32)), _spec((b6, M, 32)), _full((112, 32))])

    b7 = 64
    ec2_ws = [ec2_w0, ec2_w1, ec2_w2]
    ec2_w_specs = [_full((64, 64))] * 3
    ghs2 = []
    for depth in range(2):
        kern = _make_mid_kernel(depth, False)
        (gh,) = call(
            kern, b7,
            [out1, nb2, gm2] + ec2_ws[:depth + 1] + ghs2,
            [_spec((b7, NP, 32)), _spec((b7, M, 32)), _full((112, 32))]
            + ec2_w_specs[:depth + 1] + [_full((72, 64))] * depth,
            [((72, 64), F32)], [_full((72, 64))])
        ghs2.append(gh)

    b10 = 32
    out2, gf = call(
        _k10, b10,
        [out1, nb2, gm2, ec2_w0, ec2_w1, ec2_w2, ec2_sc_w] + ghs2,
        [_spec((b10, NP, 32)), _spec((b10, M, 32)), _full((112, 32)),
         _full((64, 64)), _full((64, 64)), _full((64, 64)), _full((64, 32))]
        + [_full((72, 64))] * 2,
        [((B, NP, 64), F32), ((104, 64), F32)],
        [_spec((b10, NP, 64)), _full((104, 64))])

    b11 = 128
    (out,) = call(
        _k11, b11,
        [out1, out2, gm2, gf, fusion_w,
         fc1_w, fc1_b.reshape(1, 128), fc2_w, fc2_b.reshape(1, 4)],
        [_spec((b11, NP, 32)), _spec((b11, NP, 64)), _full((112, 32)),
         _full((104, 64)), _full((128, 96)), _full((128, 128)),
         _full((1, 128)), _full((4, 128)), _full((1, 4))],
        [((B, 4), F32)],
        [_spec((b11, 4))])
    return out
